# baseline (speedup 1.0000x reference)
import jax
import jax.numpy as jnp
from jax.experimental import pallas as pl
from jax.experimental.pallas import tpu as pltpu

H = 64
N_GRAPHS = 16


def _decoder_kernel(z_ref, *refs):
    out_ref = refs[-1]
    it = iter(refs[:-1])
    z = z_ref[...]

    def lrelu(v):
        return jnp.where(v > 0, v, 0.15 * v)

    for li in range(3):
        W = next(it)[...]
        b = next(it)[...]
        g = next(it)[...]
        be = next(it)[...]
        m = next(it)[...]
        v = next(it)[...]
        z = lrelu(jnp.dot(z, W, preferred_element_type=jnp.float32) + b)
        z = (z - m) * jax.lax.rsqrt(v + 1e-3) * g + be
    outs = []
    for k in range(3):
        o = z
        for j in range(3):
            W = next(it)[...]
            b = next(it)[...]
            o = jnp.dot(o, W, preferred_element_type=jnp.float32) + b
            outs.append(o)
    xs = z
    for j in range(3):
        W = next(it)[...]
        b = next(it)[...]
        xs = jnp.dot(xs, W, preferred_element_type=jnp.float32) + b
    xs = jnp.abs(xs) + 1e-05
    out_ref[...] = jnp.concatenate(outs + [xs], axis=1)


def _decoder(z, p):
    args = [z]
    for li in range(3):
        args += [p[f'dec_W{li}'], p[f'dec_b{li}'], p[f'dec_g{li}'],
                 p[f'dec_be{li}'], p[f'dec_m{li}'], p[f'dec_v{li}']]
    for k in range(3):
        for j in range(3):
            args += [p[f'out{k}_W{j}'], p[f'out{k}_b{j}']]
    args += [p['sig_W0'], p['sig_b0'], p['sig_W1'], p['sig_b1'],
             p['sig_W2'], p['sig_b2']]
    return pl.pallas_call(
        _decoder_kernel,
        out_shape=jax.ShapeDtypeStruct((N_GRAPHS, 388), jnp.float32),
    )(*args)


def kernel(x, params, a_indices, i_ids):
    p = params
    N = x.shape[0]
    trans = jnp.array([0.0, 0.0, -200.0, 10000.0, 0.0], dtype=jnp.float32)
    scale = jnp.array([100.0, 100.0, 100.0, 2500.0, 0.25], dtype=jnp.float32)
    x = (x - trans) / scale
    send = a_indices[:, 0].astype(jnp.int32)
    recv = a_indices[:, 1].astype(jnp.int32)
    i_ids = i_ids.astype(jnp.int32)
    diff = x[recv] - x[send]
    dists = jnp.sqrt(jnp.sum(diff[:, :3] ** 2, axis=1))
    den = jnp.where(dists == 0, 1.0, dists)[:, None]
    vects = jnp.where(dists[:, None] != 0, diff[:, :3] / den, 0.0)
    e = jnp.concatenate([diff[:, 3:], dists[:, None], vects], axis=1)
    e = (e - p['bn_e_mean']) / jnp.sqrt(p['bn_e_var'] + 1e-3) * p['bn_e_gamma'] + p['bn_e_beta']
    m = jnp.concatenate([x[send], x[recv], e], axis=1)
    m = jax.nn.relu(m @ p['mp_msg_W1'] + p['mp_msg_b1'])
    m = jax.nn.relu(m @ p['mp_msg_W2'] + p['mp_msg_b2'])
    seg = send
    cnt = jax.ops.segment_sum(jnp.ones(seg.shape, jnp.float32), seg, num_segments=N)
    cnt_safe = jnp.maximum(cnt, 1.0)[:, None]
    has = (cnt > 0)[:, None]
    mn = jnp.where(has, jax.ops.segment_min(m, seg, num_segments=N), 0.0)
    mx = jnp.where(has, jax.ops.segment_max(m, seg, num_segments=N), 0.0)
    mean = jax.ops.segment_sum(m, seg, num_segments=N) / cnt_safe
    mean2 = jax.ops.segment_sum(m ** 2, seg, num_segments=N) / cnt_safe
    var = mean2 - mean ** 2
    emb = jnp.concatenate([mn, mx, mean, var], axis=1)
    h = jax.nn.relu(emb @ p['mp_upd_W1'] + p['mp_upd_b1'])
    h = jax.nn.relu(h @ p['mp_upd_W2'] + p['mp_upd_b2'])

    def gsage(xx, W, b):
        agg = jax.ops.segment_sum(xx[recv], send, num_segments=N) / cnt_safe
        out = jnp.concatenate([xx, agg], axis=1) @ W + b
        norm = jnp.sqrt(jnp.maximum(jnp.sum(out ** 2, axis=-1, keepdims=True), 1e-12))
        return jax.nn.relu(out / norm)

    h = gsage(h, p['gs1_W'], p['gs1_b'])
    h = gsage(h, p['gs2_W'], p['gs2_b'])
    G = N_GRAPHS
    gcnt = jnp.maximum(jax.ops.segment_sum(jnp.ones(i_ids.shape, jnp.float32), i_ids, num_segments=G), 1.0)[:, None]
    s = jax.ops.segment_sum(h, i_ids, num_segments=G)
    x1 = jax.ops.segment_max(h, i_ids, num_segments=G)
    x2 = s / gcnt
    x4 = -jax.ops.segment_max(-h, i_ids, num_segments=G)
    z = jnp.concatenate([x1, x2, s, x4], axis=1)
    return _decoder(z, p)


# speedup vs baseline: 1.3086x; 1.3086x over previous
import functools

import jax
import jax.numpy as jnp
from jax import lax
from jax.experimental import pallas as pl
from jax.experimental.pallas import tpu as pltpu
from jax.experimental.pallas import tpu_sc as plsc

H = 64
N_GRAPHS = 16

NC, NS = 2, 16
NW = NC * NS
BATCH = 128
SUB = 16
CHUNK = BATCH * SUB


def _sc_gather_segsum(table, ig2, is2, zeros, n_chunks, d):
    npad = zeros.shape[0]
    stripe = npad // NS
    mesh = plsc.VectorSubcoreMesh(core_axis_name="c", subcore_axis_name="s")
    iters = (n_chunks + NW - 1) // NW

    @functools.partial(
        pl.kernel,
        out_type=jax.ShapeDtypeStruct((NC, npad, d), jnp.float32),
        mesh=mesh,
        scratch_types=[
            pltpu.VMEM((SUB, BATCH), jnp.int32),
            pltpu.VMEM((SUB, BATCH), jnp.int32),
            pltpu.VMEM((CHUNK, d), jnp.float32),
            pltpu.VMEM_SHARED((npad, d), jnp.float32),
            pltpu.SemaphoreType.DMA,
        ],
        compiler_params=pltpu.CompilerParams(use_tc_tiling_on_sc=False),
    )
    def k(table_h, ig_h, is_h, z_h, out_h, igv, isv, rows, acc, sem):
        c = lax.axis_index("c")
        s = lax.axis_index("s")
        pltpu.sync_copy(z_h.at[pl.ds(s * stripe, stripe)],
                        acc.at[pl.ds(s * stripe, stripe)])
        plsc.subcore_barrier()
        wid = s * NC + c

        def body(i, _):
            g = i * NW + wid

            @pl.when(g < n_chunks)
            def _():
                pltpu.sync_copy(ig_h.at[pl.ds(g * SUB, SUB)], igv)
                pltpu.sync_copy(is_h.at[pl.ds(g * SUB, SUB)], isv)
                copies = []
                for j in range(SUB):
                    copies.append(pltpu.async_copy(
                        table_h.at[igv.at[j]],
                        rows.at[pl.ds(j * BATCH, BATCH)], sem))
                for cp in copies:
                    cp.wait()
                for j in range(SUB):
                    pltpu.sync_copy(rows.at[pl.ds(j * BATCH, BATCH)],
                                    acc.at[isv.at[j]], add=True)
            return 0

        lax.fori_loop(0, iters, body, 0)
        plsc.subcore_barrier()
        pltpu.sync_copy(acc.at[pl.ds(s * stripe, stripe)],
                        out_h.at[c, pl.ds(s * stripe, stripe)])

    return k(table, ig2, is2, zeros)


def _decoder_kernel(z_ref, *refs):
    out_ref = refs[-1]
    it = iter(refs[:-1])
    z = z_ref[...]

    def lrelu(v):
        return jnp.where(v > 0, v, 0.15 * v)

    for li in range(3):
        W = next(it)[...]
        b = next(it)[...]
        g = next(it)[...]
        be = next(it)[...]
        m = next(it)[...]
        v = next(it)[...]
        z = lrelu(jnp.dot(z, W, preferred_element_type=jnp.float32) + b)
        z = (z - m) * jax.lax.rsqrt(v + 1e-3) * g + be
    outs = []
    for k in range(3):
        o = z
        for j in range(3):
            W = next(it)[...]
            b = next(it)[...]
            o = jnp.dot(o, W, preferred_element_type=jnp.float32) + b
            outs.append(o)
    xs = z
    for j in range(3):
        W = next(it)[...]
        b = next(it)[...]
        xs = jnp.dot(xs, W, preferred_element_type=jnp.float32) + b
    xs = jnp.abs(xs) + 1e-05
    out_ref[...] = jnp.concatenate(outs + [xs], axis=1)


def _decoder(z, p):
    args = [z]
    for li in range(3):
        args += [p[f'dec_W{li}'], p[f'dec_b{li}'], p[f'dec_g{li}'],
                 p[f'dec_be{li}'], p[f'dec_m{li}'], p[f'dec_v{li}']]
    for k in range(3):
        for j in range(3):
            args += [p[f'out{k}_W{j}'], p[f'out{k}_b{j}']]
    args += [p['sig_W0'], p['sig_b0'], p['sig_W1'], p['sig_b1'],
             p['sig_W2'], p['sig_b2']]
    return pl.pallas_call(
        _decoder_kernel,
        out_shape=jax.ShapeDtypeStruct((N_GRAPHS, 388), jnp.float32),
    )(*args)


def kernel(x, params, a_indices, i_ids):
    p = params
    N = x.shape[0]
    trans = jnp.array([0.0, 0.0, -200.0, 10000.0, 0.0], dtype=jnp.float32)
    scale = jnp.array([100.0, 100.0, 100.0, 2500.0, 0.25], dtype=jnp.float32)
    x = (x - trans) / scale
    send = a_indices[:, 0].astype(jnp.int32)
    recv = a_indices[:, 1].astype(jnp.int32)
    i_ids = i_ids.astype(jnp.int32)
    diff = x[recv] - x[send]
    dists = jnp.sqrt(jnp.sum(diff[:, :3] ** 2, axis=1))
    den = jnp.where(dists == 0, 1.0, dists)[:, None]
    vects = jnp.where(dists[:, None] != 0, diff[:, :3] / den, 0.0)
    e = jnp.concatenate([diff[:, 3:], dists[:, None], vects], axis=1)
    e = (e - p['bn_e_mean']) / jnp.sqrt(p['bn_e_var'] + 1e-3) * p['bn_e_gamma'] + p['bn_e_beta']
    m = jnp.concatenate([x[send], x[recv], e], axis=1)
    m = jax.nn.relu(m @ p['mp_msg_W1'] + p['mp_msg_b1'])
    m = jax.nn.relu(m @ p['mp_msg_W2'] + p['mp_msg_b2'])
    seg = send
    cnt = jax.ops.segment_sum(jnp.ones(seg.shape, jnp.float32), seg, num_segments=N)
    cnt_safe = jnp.maximum(cnt, 1.0)[:, None]
    has = (cnt > 0)[:, None]
    mn = jnp.where(has, jax.ops.segment_min(m, seg, num_segments=N), 0.0)
    mx = jnp.where(has, jax.ops.segment_max(m, seg, num_segments=N), 0.0)
    mean = jax.ops.segment_sum(m, seg, num_segments=N) / cnt_safe
    mean2 = jax.ops.segment_sum(m ** 2, seg, num_segments=N) / cnt_safe
    var = mean2 - mean ** 2
    emb = jnp.concatenate([mn, mx, mean, var], axis=1)
    h = jax.nn.relu(emb @ p['mp_upd_W1'] + p['mp_upd_b1'])
    h = jax.nn.relu(h @ p['mp_upd_W2'] + p['mp_upd_b2'])

    E = send.shape[0]
    n_chunks = (E + CHUNK - 1) // CHUNK
    e_pad = n_chunks * CHUNK
    npad = ((N + NS * 8 - 1) // (NS * 8)) * NS * 8
    recv_pad = jnp.concatenate([recv, jnp.zeros((e_pad - E,), jnp.int32)])
    send_pad = jnp.concatenate([send, jnp.full((e_pad - E,), N, jnp.int32)])
    ig2 = recv_pad.reshape(n_chunks * SUB, BATCH)
    is2 = send_pad.reshape(n_chunks * SUB, BATCH)

    def gsage(xx, W, b):
        d = xx.shape[1]
        dpad = ((d + 15) // 16) * 16
        if dpad != d:
            xxp = jnp.concatenate(
                [xx, jnp.zeros((N, dpad - d), jnp.float32)], axis=1)
        else:
            xxp = xx
        blocks = []
        for k0 in range(0, dpad, 16):
            zeros = jnp.zeros((npad, 16), jnp.float32)
            part = _sc_gather_segsum(xxp[:, k0:k0 + 16], ig2, is2, zeros,
                                     n_chunks, 16)
            blocks.append(part[0, :N] + part[1, :N])
        agg = jnp.concatenate(blocks, axis=1)[:, :d] / cnt_safe
        out = jnp.concatenate([xx, agg], axis=1) @ W + b
        norm = jnp.sqrt(jnp.maximum(jnp.sum(out ** 2, axis=-1, keepdims=True), 1e-12))
        return jax.nn.relu(out / norm)

    h = gsage(h, p['gs1_W'], p['gs1_b'])
    h = gsage(h, p['gs2_W'], p['gs2_b'])
    G = N_GRAPHS
    gcnt = jnp.maximum(jax.ops.segment_sum(jnp.ones(i_ids.shape, jnp.float32), i_ids, num_segments=G), 1.0)[:, None]
    s = jax.ops.segment_sum(h, i_ids, num_segments=G)
    x1 = jax.ops.segment_max(h, i_ids, num_segments=G)
    x2 = s / gcnt
    x4 = -jax.ops.segment_max(-h, i_ids, num_segments=G)
    z = jnp.concatenate([x1, x2, s, x4], axis=1)
    return _decoder(z, p)


# speedup vs baseline: 1.3269x; 1.0140x over previous
import functools

import jax
import jax.numpy as jnp
from jax import lax
from jax.experimental import pallas as pl
from jax.experimental.pallas import tpu as pltpu
from jax.experimental.pallas import tpu_sc as plsc

H = 64
N_GRAPHS = 16

NC, NS = 2, 16
NW = NC * NS
BATCH = 128
SUB = 16
CHUNK = BATCH * SUB


def _sc_gather_segsum(table, ig2, is2, zeros, n_chunks, d):
    npad = zeros.shape[0]
    stripe = npad // NS
    mesh = plsc.VectorSubcoreMesh(core_axis_name="c", subcore_axis_name="s")
    iters = (n_chunks + NW - 1) // NW

    @functools.partial(
        pl.kernel,
        out_type=jax.ShapeDtypeStruct((NC, npad, d), jnp.float32),
        mesh=mesh,
        scratch_types=[
            pltpu.VMEM((2 * SUB, BATCH), jnp.int32),
            pltpu.VMEM((2 * SUB, BATCH), jnp.int32),
            pltpu.VMEM((CHUNK, d), jnp.float32),
            pltpu.VMEM((CHUNK, d), jnp.float32),
            pltpu.VMEM_SHARED((npad, d), jnp.float32),
            pltpu.SemaphoreType.DMA,
            pltpu.SemaphoreType.DMA,
        ],
        compiler_params=pltpu.CompilerParams(use_tc_tiling_on_sc=False),
    )
    def k(table_h, ig_h, is_h, z_h, out_h, igv, isv, rowsa, rowsb, acc,
          semg, sema):
        c = lax.axis_index("c")
        s = lax.axis_index("s")
        pltpu.sync_copy(z_h.at[pl.ds(s * stripe, stripe)],
                        acc.at[pl.ds(s * stripe, stripe)])
        plsc.subcore_barrier()
        wid = s * NC + c

        def body(i, _):
            g0 = (2 * i) * NW + wid
            g1 = (2 * i + 1) * NW + wid

            def fire_gathers(g, rows, half):
                pltpu.sync_copy(ig_h.at[pl.ds(g * SUB, SUB)],
                                igv.at[pl.ds(half * SUB, SUB)])
                pltpu.sync_copy(is_h.at[pl.ds(g * SUB, SUB)],
                                isv.at[pl.ds(half * SUB, SUB)])
                return [pltpu.async_copy(
                    table_h.at[igv.at[half * SUB + j]],
                    rows.at[pl.ds(j * BATCH, BATCH)], semg)
                    for j in range(SUB)]

            def fire_adds(rows, half):
                return [pltpu.async_copy(
                    rows.at[pl.ds(j * BATCH, BATCH)],
                    acc.at[isv.at[half * SUB + j]], sema, add=True)
                    for j in range(SUB)]

            @pl.when(g0 < n_chunks)
            def _():
                ga = fire_gathers(g0, rowsa, 0)

                @pl.when(g1 < n_chunks)
                def _():
                    gb = fire_gathers(g1, rowsb, 1)
                    for cp in ga:
                        cp.wait()
                    aa = fire_adds(rowsa, 0)
                    for cp in gb:
                        cp.wait()
                    ab = fire_adds(rowsb, 1)
                    for cp in aa + ab:
                        cp.wait()

                @pl.when(g1 >= n_chunks)
                def _():
                    for cp in ga:
                        cp.wait()
                    aa = fire_adds(rowsa, 0)
                    for cp in aa:
                        cp.wait()
            return 0

        lax.fori_loop(0, (iters + 1) // 2, body, 0)
        plsc.subcore_barrier()
        pltpu.sync_copy(acc.at[pl.ds(s * stripe, stripe)],
                        out_h.at[c, pl.ds(s * stripe, stripe)])

    return k(table, ig2, is2, zeros)


def _decoder_kernel(z_ref, *refs):
    out_ref = refs[-1]
    it = iter(refs[:-1])
    z = z_ref[...]

    def lrelu(v):
        return jnp.where(v > 0, v, 0.15 * v)

    for li in range(3):
        W = next(it)[...]
        b = next(it)[...]
        g = next(it)[...]
        be = next(it)[...]
        m = next(it)[...]
        v = next(it)[...]
        z = lrelu(jnp.dot(z, W, preferred_element_type=jnp.float32) + b)
        z = (z - m) * jax.lax.rsqrt(v + 1e-3) * g + be
    outs = []
    for k in range(3):
        o = z
        for j in range(3):
            W = next(it)[...]
            b = next(it)[...]
            o = jnp.dot(o, W, preferred_element_type=jnp.float32) + b
            outs.append(o)
    xs = z
    for j in range(3):
        W = next(it)[...]
        b = next(it)[...]
        xs = jnp.dot(xs, W, preferred_element_type=jnp.float32) + b
    xs = jnp.abs(xs) + 1e-05
    out_ref[...] = jnp.concatenate(outs + [xs], axis=1)


def _decoder(z, p):
    args = [z]
    for li in range(3):
        args += [p[f'dec_W{li}'], p[f'dec_b{li}'], p[f'dec_g{li}'],
                 p[f'dec_be{li}'], p[f'dec_m{li}'], p[f'dec_v{li}']]
    for k in range(3):
        for j in range(3):
            args += [p[f'out{k}_W{j}'], p[f'out{k}_b{j}']]
    args += [p['sig_W0'], p['sig_b0'], p['sig_W1'], p['sig_b1'],
             p['sig_W2'], p['sig_b2']]
    return pl.pallas_call(
        _decoder_kernel,
        out_shape=jax.ShapeDtypeStruct((N_GRAPHS, 388), jnp.float32),
    )(*args)


def kernel(x, params, a_indices, i_ids):
    p = params
    N = x.shape[0]
    trans = jnp.array([0.0, 0.0, -200.0, 10000.0, 0.0], dtype=jnp.float32)
    scale = jnp.array([100.0, 100.0, 100.0, 2500.0, 0.25], dtype=jnp.float32)
    x = (x - trans) / scale
    send = a_indices[:, 0].astype(jnp.int32)
    recv = a_indices[:, 1].astype(jnp.int32)
    i_ids = i_ids.astype(jnp.int32)
    diff = x[recv] - x[send]
    dists = jnp.sqrt(jnp.sum(diff[:, :3] ** 2, axis=1))
    den = jnp.where(dists == 0, 1.0, dists)[:, None]
    vects = jnp.where(dists[:, None] != 0, diff[:, :3] / den, 0.0)
    e = jnp.concatenate([diff[:, 3:], dists[:, None], vects], axis=1)
    e = (e - p['bn_e_mean']) / jnp.sqrt(p['bn_e_var'] + 1e-3) * p['bn_e_gamma'] + p['bn_e_beta']
    m = jnp.concatenate([x[send], x[recv], e], axis=1)
    m = jax.nn.relu(m @ p['mp_msg_W1'] + p['mp_msg_b1'])
    m = jax.nn.relu(m @ p['mp_msg_W2'] + p['mp_msg_b2'])
    seg = send
    cnt = jax.ops.segment_sum(jnp.ones(seg.shape, jnp.float32), seg, num_segments=N)
    cnt_safe = jnp.maximum(cnt, 1.0)[:, None]
    has = (cnt > 0)[:, None]
    mn = jnp.where(has, jax.ops.segment_min(m, seg, num_segments=N), 0.0)
    mx = jnp.where(has, jax.ops.segment_max(m, seg, num_segments=N), 0.0)
    mean = jax.ops.segment_sum(m, seg, num_segments=N) / cnt_safe
    mean2 = jax.ops.segment_sum(m ** 2, seg, num_segments=N) / cnt_safe
    var = mean2 - mean ** 2
    emb = jnp.concatenate([mn, mx, mean, var], axis=1)
    h = jax.nn.relu(emb @ p['mp_upd_W1'] + p['mp_upd_b1'])
    h = jax.nn.relu(h @ p['mp_upd_W2'] + p['mp_upd_b2'])

    E = send.shape[0]
    n_chunks = (E + CHUNK - 1) // CHUNK
    e_pad = n_chunks * CHUNK
    npad = ((N + NS * 8 - 1) // (NS * 8)) * NS * 8
    recv_pad = jnp.concatenate([recv, jnp.zeros((e_pad - E,), jnp.int32)])
    send_pad = jnp.concatenate([send, jnp.full((e_pad - E,), N, jnp.int32)])
    ig2 = recv_pad.reshape(n_chunks * SUB, BATCH)
    is2 = send_pad.reshape(n_chunks * SUB, BATCH)

    def gsage(xx, W, b):
        d = xx.shape[1]
        dpad = ((d + 15) // 16) * 16
        if dpad != d:
            xxp = jnp.concatenate(
                [xx, jnp.zeros((N, dpad - d), jnp.float32)], axis=1)
        else:
            xxp = xx
        blocks = []
        for k0 in range(0, dpad, 16):
            zeros = jnp.zeros((npad, 16), jnp.float32)
            part = _sc_gather_segsum(xxp[:, k0:k0 + 16], ig2, is2, zeros,
                                     n_chunks, 16)
            blocks.append(part[0, :N] + part[1, :N])
        agg = jnp.concatenate(blocks, axis=1)[:, :d] / cnt_safe
        out = jnp.concatenate([xx, agg], axis=1) @ W + b
        norm = jnp.sqrt(jnp.maximum(jnp.sum(out ** 2, axis=-1, keepdims=True), 1e-12))
        return jax.nn.relu(out / norm)

    h = gsage(h, p['gs1_W'], p['gs1_b'])
    h = gsage(h, p['gs2_W'], p['gs2_b'])
    G = N_GRAPHS
    gcnt = jnp.maximum(jax.ops.segment_sum(jnp.ones(i_ids.shape, jnp.float32), i_ids, num_segments=G), 1.0)[:, None]
    s = jax.ops.segment_sum(h, i_ids, num_segments=G)
    x1 = jax.ops.segment_max(h, i_ids, num_segments=G)
    x2 = s / gcnt
    x4 = -jax.ops.segment_max(-h, i_ids, num_segments=G)
    z = jnp.concatenate([x1, x2, s, x4], axis=1)
    return _decoder(z, p)
